# baseline (speedup 1.0000x reference)
import functools

import jax
import jax.numpy as jnp
from jax import lax
from jax.experimental import pallas as pl
from jax.experimental.pallas import tpu as pltpu
from jax.experimental.pallas import tpu_sc as plsc

EMBED_DIM = 64
BATCH = 16384
LANES = 16
NUM_CORES = 2
NUM_SUBCORES = 16
NW = NUM_CORES * NUM_SUBCORES
B_PER_W = BATCH // NW
CHUNK = 128
NCHUNK = B_PER_W // CHUNK
GROUPS = CHUNK // LANES
KSLICES = EMBED_DIM // LANES


def _sc_body(src_hbm, tgt_hbm, emb_hbm, gat_hbm, out_hbm,
             sidx, tidx, wt, ws, gt, gs, tbuf, outv, sem):
    c = lax.axis_index("c")
    s = lax.axis_index("s")
    wid = s * NUM_CORES + c
    base = wid * B_PER_W
    lane = lax.iota(jnp.int32, LANES)

    def chunk_body(ci, carry):
        off = base + ci * CHUNK
        pltpu.sync_copy(src_hbm.at[pl.ds(off, CHUNK)], sidx)
        pltpu.sync_copy(tgt_hbm.at[pl.ds(off, CHUNK)], tidx)
        cps = [
            pltpu.async_copy(emb_hbm.at[tidx], wt, sem),
            pltpu.async_copy(emb_hbm.at[sidx], ws, sem),
            pltpu.async_copy(gat_hbm.at[tidx], gt, sem),
            pltpu.async_copy(gat_hbm.at[sidx], gs, sem),
        ]
        for cp in cps:
            cp.wait()

        def group_body(g, gcarry):
            r0 = g * LANES
            acc = jnp.zeros((LANES,), jnp.float32)
            for e in range(LANES):
                r = r0 + e
                p = jnp.zeros((LANES,), jnp.float32)
                for k in range(KSLICES):
                    sl = pl.ds(k * LANES, LANES)
                    gv = gt[r, sl] + gs[r, sl]
                    gate = 1.0 / (1.0 + jnp.exp(-gv))
                    p = p + gate * (wt[r, sl] - ws[r, sl])
                tot = jnp.sum(p)
                acc = jnp.where(lane == e, tot, acc)
            outv[pl.ds(ci * CHUNK + g * LANES, LANES)] = acc
            return gcarry

        lax.fori_loop(0, GROUPS, group_body, 0)
        return carry

    lax.fori_loop(0, NCHUNK, chunk_body, 0)
    pltpu.sync_copy(outv, out_hbm.at[pl.ds(base, B_PER_W)])


@jax.jit
def kernel(source_nodes, target_nodes, node_embeddings, gates):
    mesh = plsc.VectorSubcoreMesh(core_axis_name="c", subcore_axis_name="s")
    k = pl.kernel(
        _sc_body,
        mesh=mesh,
        compiler_params=pltpu.CompilerParams(
            needs_layout_passes=False, use_tc_tiling_on_sc=False
        ),
        out_type=jax.ShapeDtypeStruct((BATCH,), jnp.float32),
        scratch_types=[
            pltpu.VMEM((CHUNK,), jnp.int32),
            pltpu.VMEM((CHUNK,), jnp.int32),
            pltpu.VMEM((CHUNK, EMBED_DIM), jnp.float32),
            pltpu.VMEM((CHUNK, EMBED_DIM), jnp.float32),
            pltpu.VMEM((CHUNK, EMBED_DIM), jnp.float32),
            pltpu.VMEM((CHUNK, EMBED_DIM), jnp.float32),
            pltpu.VMEM((LANES * LANES,), jnp.float32),
            pltpu.VMEM((B_PER_W,), jnp.float32),
            pltpu.SemaphoreType.DMA,
        ],
    )
    return k(
        jnp.asarray(source_nodes, jnp.int32),
        jnp.asarray(target_nodes, jnp.int32),
        node_embeddings,
        gates,
    )


# speedup vs baseline: 2.2645x; 2.2645x over previous
import functools

import jax
import jax.numpy as jnp
from jax import lax
from jax.experimental import pallas as pl
from jax.experimental.pallas import tpu as pltpu
from jax.experimental.pallas import tpu_sc as plsc

NUM_NODES = 1000000
EMBED_DIM = 64
BATCH = 16384
LANES = 16
NUM_CORES = 2
NUM_SUBCORES = 16
NW = NUM_CORES * NUM_SUBCORES
B_PER_W = BATCH // NW
STEP = LANES
NSTEP = B_PER_W // STEP
TILE_ROWS = 8
NTILES = NUM_NODES // TILE_ROWS


def _sc_body(src_hbm, tgt_hbm, emb_hbm, gat_hbm, out_hbm,
             sidx, tidx, ts_t, ts_s, tg_t, tg_s, it_t, it_s, outv, sem):
    c = lax.axis_index("c")
    s = lax.axis_index("s")
    wid = s * NUM_CORES + c
    base = wid * B_PER_W
    lane = lax.iota(jnp.int32, LANES)

    pltpu.sync_copy(src_hbm.at[pl.ds(base, B_PER_W)], sidx)
    pltpu.sync_copy(tgt_hbm.at[pl.ds(base, B_PER_W)], tidx)

    def step_body(st, carry):
        iv_t = tidx[pl.ds(st * STEP, STEP)]
        iv_s = sidx[pl.ds(st * STEP, STEP)]
        til_t = lax.shift_right_logical(iv_t, 3)
        til_s = lax.shift_right_logical(iv_s, 3)
        row_t = lax.bitwise_and(iv_t, 7)
        row_s = lax.bitwise_and(iv_s, 7)
        cps = []
        for j in range(STEP):
            m = lane == j
            tt = jnp.sum(jnp.where(m, til_t, 0))
            rt = jnp.sum(jnp.where(m, row_t, 0))
            ts = jnp.sum(jnp.where(m, til_s, 0))
            rs = jnp.sum(jnp.where(m, row_s, 0))
            cps.append(pltpu.async_copy(emb_hbm.at[tt, rt], ts_t.at[j], sem))
            cps.append(pltpu.async_copy(emb_hbm.at[ts, rs], ts_s.at[j], sem))
            cps.append(pltpu.async_copy(gat_hbm.at[tt, rt], tg_t.at[j], sem))
            cps.append(pltpu.async_copy(gat_hbm.at[ts, rs], tg_s.at[j], sem))
        for cp in cps:
            cp.wait()

        acc = jnp.zeros((LANES,), jnp.float32)
        for j in range(STEP):
            p = jnp.zeros((LANES,), jnp.float32)
            for k in range(EMBED_DIM // LANES):
                sl = pl.ds(k * LANES, LANES)
                gv = tg_t[j, sl] + tg_s[j, sl]
                gate = 1.0 / (1.0 + jnp.exp(-gv))
                p = p + gate * (ts_t[j, sl] - ts_s[j, sl])
            acc = jnp.where(lane == j, jnp.sum(p), acc)
        outv[pl.ds(st * STEP, STEP)] = acc
        return carry

    lax.fori_loop(0, NSTEP, step_body, 0)
    pltpu.sync_copy(outv, out_hbm.at[pl.ds(base, B_PER_W)])


@jax.jit
def kernel(source_nodes, target_nodes, node_embeddings, gates):
    mesh = plsc.VectorSubcoreMesh(core_axis_name="c", subcore_axis_name="s")
    k = pl.kernel(
        _sc_body,
        mesh=mesh,
        compiler_params=pltpu.CompilerParams(needs_layout_passes=False),
        out_type=jax.ShapeDtypeStruct((BATCH,), jnp.float32),
        scratch_types=[
            pltpu.VMEM((B_PER_W,), jnp.int32),
            pltpu.VMEM((B_PER_W,), jnp.int32),
            pltpu.VMEM((STEP, EMBED_DIM), jnp.float32),
            pltpu.VMEM((STEP, EMBED_DIM), jnp.float32),
            pltpu.VMEM((STEP, EMBED_DIM), jnp.float32),
            pltpu.VMEM((STEP, EMBED_DIM), jnp.float32),
            pltpu.VMEM((STEP,), jnp.int32),
            pltpu.VMEM((STEP,), jnp.int32),
            pltpu.VMEM((B_PER_W,), jnp.float32),
            pltpu.SemaphoreType.DMA,
        ],
    )
    emb3 = node_embeddings.reshape(NTILES, TILE_ROWS, EMBED_DIM)
    gat3 = gates.reshape(NTILES, TILE_ROWS, EMBED_DIM)
    return k(
        jnp.asarray(source_nodes, jnp.int32),
        jnp.asarray(target_nodes, jnp.int32),
        emb3,
        gat3,
    )
